# baseline (speedup 1.0000x reference)
import functools

import jax
import jax.numpy as jnp
from jax import lax
from jax.experimental import pallas as pl
from jax.experimental.pallas import tpu as pltpu
from jax.experimental.pallas import tpu_sc as plsc

T, C, E = 2048, 768, 8
H = 4 * C
RB = 256
NB = (2 * T) // RB + E
PAD = NB * RB
NW = 32
TPW = T // NW
GRP = 128
NG = T // GRP
NEG = -3.0e38


def _router_body(x_ref, wr_ref, br_ref, wn_ref, bn_ref, nz_ref,
                 pos0_ref, pos1_ref, g0_ref, g1_ref, bexp_ref, cb_ref):
    xf = x_ref[...]
    nt = (((1,), (1,)), ((), ()))
    logits = lax.dot_general(xf, wr_ref[...], nt,
                             preferred_element_type=jnp.float32) + br_ref[...]
    nlog = lax.dot_general(xf, wn_ref[...], nt,
                           preferred_element_type=jnp.float32) + bn_ref[...]
    noisy = logits + nz_ref[...] * jax.nn.softplus(nlog)

    ia = lax.broadcasted_iota(jnp.int32, (E, E), 0)
    ib = lax.broadcasted_iota(jnp.int32, (E, E), 1)
    u8s = (ia < ib).astype(jnp.float32)
    u8i = (ia <= ib).astype(jnp.float32)
    nn = (((1,), (0,)), ((), ()))

    v0 = jnp.max(noisy, axis=1, keepdims=True)
    eq0 = (noisy == v0).astype(jnp.float32)
    p0 = lax.dot_general(eq0, u8s, nn, preferred_element_type=jnp.float32)
    oh0 = eq0 * (p0 == 0).astype(jnp.float32)
    masked = jnp.where(oh0 > 0, NEG, noisy)
    v1 = jnp.max(masked, axis=1, keepdims=True)
    eq1 = (masked == v1).astype(jnp.float32)
    p1 = lax.dot_general(eq1, u8s, nn, preferred_element_type=jnp.float32)
    oh1 = eq1 * (p1 == 0).astype(jnp.float32)

    e1g = jnp.exp(v1 - v0)
    den = 1.0 + e1g
    g0 = 1.0 / den
    g1 = e1g / den

    msk = oh0 + oh1
    ri = lax.broadcasted_iota(jnp.int32, (GRP, GRP), 0)
    rj = lax.broadcasted_iota(jnp.int32, (GRP, GRP), 1)
    lg = (rj < ri).astype(jnp.float32)
    acc = jnp.zeros((1, E), dtype=jnp.float32)
    parts = []
    for g in range(NG):
        mg = msk[g * GRP:(g + 1) * GRP]
        pg = lax.dot_general(lg, mg, nn, preferred_element_type=jnp.float32)
        parts.append(pg + acc)
        acc = acc + pg[GRP - 1:GRP] + mg[GRP - 1:GRP]
    rank = jnp.concatenate(parts, axis=0)
    counts = acc

    nblk = jnp.floor((counts + (RB - 1)) * (1.0 / RB))
    cumb = lax.dot_general(nblk, u8i, nn,
                           preferred_element_type=jnp.float32)
    offs = (cumb - nblk) * float(RB)

    posf = rank + offs
    pos0_ref[...] = jnp.sum(posf * oh0, axis=1, keepdims=True).astype(jnp.int32)
    pos1_ref[...] = jnp.sum(posf * oh1, axis=1, keepdims=True).astype(jnp.int32)
    g0_ref[...] = jnp.broadcast_to(g0, (T, 16))
    g1_ref[...] = jnp.broadcast_to(g1, (T, 16))

    biota = lax.broadcasted_iota(jnp.int32, (128, E), 0)
    cumb_i = cumb.astype(jnp.int32)
    bexp = jnp.sum((cumb_i <= biota).astype(jnp.int32), axis=1, keepdims=True)
    bexp_ref[...] = jnp.minimum(bexp, E - 1)
    cb_ref[...] = cumb_i


HH = H // 2


def _ffn_outer(sc_ref, xg_hbm, w1_hbm, b1_hbm, w2_hbm, b2_hbm, lg_hbm,
               lb_hbm, yln_hbm):
    nbtot = sc_ref[NB]

    def inner(idx, xg_ref, w1_ref, b1_ref, w2_ref, b2_ref, lg_ref, lb_ref,
              out_ref):
        _, s = idx
        nt = (((1,), (1,)), ((), ()))
        xb = xg_ref[...]
        h = lax.dot_general(xb, w1_ref[0], nt,
                            preferred_element_type=jnp.float32)
        h = h + b1_ref[0]
        h = 0.5 * h * (1.0 + lax.erf(h * 0.7071067811865476))
        part = lax.dot_general(h, w2_ref[0], nt,
                               preferred_element_type=jnp.float32)

        @pl.when(s == 0)
        def _():
            out_ref[...] = part

        @pl.when(s == 1)
        def _():
            z = xb + (out_ref[...] + part + b2_ref[0])
            mu = jnp.mean(z, axis=1, keepdims=True)
            zc = z - mu
            var = jnp.mean(zc * zc, axis=1, keepdims=True)
            out_ref[...] = zc * lax.rsqrt(var + 1e-5) * lg_ref[0] + lb_ref[0]

    wbuf = pl.Buffered(buffer_count=4, use_lookahead=True)
    pltpu.emit_pipeline(
        inner,
        grid=(nbtot, 2),
        in_specs=[
            pl.BlockSpec((RB, C), lambda i, s: (i, 0)),
            pl.BlockSpec((1, HH, C), lambda i, s: (sc_ref[i], s, 0),
                         pipeline_mode=wbuf),
            pl.BlockSpec((1, 1, HH), lambda i, s: (sc_ref[i], 0, s)),
            pl.BlockSpec((1, C, HH), lambda i, s: (sc_ref[i], 0, s),
                         pipeline_mode=wbuf),
            pl.BlockSpec((1, 1, C), lambda i, s: (sc_ref[i], 0, 0)),
            pl.BlockSpec((1, 1, C), lambda i, s: (sc_ref[i], 0, 0)),
            pl.BlockSpec((1, 1, C), lambda i, s: (sc_ref[i], 0, 0)),
        ],
        out_specs=[pl.BlockSpec((RB, C), lambda i, s: (i, 0))],
        _explicit_indices=True,
    )(xg_hbm, w1_hbm, b1_hbm, w2_hbm, b2_hbm, lg_hbm, lb_hbm, yln_hbm)


def _dispatch_body(x_hbm, p0_hbm, p1_hbm, xg_hbm, idx0, idx1, rows, s0, s1):
    wid = lax.axis_index("s") * 2 + lax.axis_index("c")
    base = wid * TPW
    pltpu.sync_copy(p0_hbm.at[wid], idx0)
    pltpu.sync_copy(p1_hbm.at[wid], idx1)
    pltpu.sync_copy(x_hbm.at[pl.ds(base, TPW)], rows)
    d0 = pltpu.async_copy(rows, xg_hbm.at[idx0], s0)
    d1 = pltpu.async_copy(rows, xg_hbm.at[idx1], s1)
    d0.wait()
    d1.wait()


def _combine_body(yln_hbm, p0_hbm, p1_hbm, g0_hbm, g1_hbm, out_hbm,
                  idx0, idx1, r0, r1, gv0, gv1, s0, s1):
    wid = lax.axis_index("s") * 2 + lax.axis_index("c")
    base = wid * TPW
    pltpu.sync_copy(p0_hbm.at[wid], idx0)
    pltpu.sync_copy(p1_hbm.at[wid], idx1)
    pltpu.sync_copy(g0_hbm.at[wid], gv0)
    pltpu.sync_copy(g1_hbm.at[wid], gv1)
    d0 = pltpu.async_copy(yln_hbm.at[idx0], r0, s0)
    d1 = pltpu.async_copy(yln_hbm.at[idx1], r1, s1)
    d0.wait()
    d1.wait()

    def tok(i, carry):
        ga = gv0[i]
        gb = gv1[i]
        for cc in range(C // 16):
            sl = pl.ds(cc * 16, 16)
            r0[i, sl] = ga * r0[i, sl] + gb * r1[i, sl]
        return carry

    lax.fori_loop(0, TPW, tok, 0)
    pltpu.sync_copy(r0, out_hbm.at[pl.ds(base, TPW)])


@functools.lru_cache(maxsize=1)
def _sc_kernels():
    mesh = plsc.VectorSubcoreMesh(core_axis_name="c", subcore_axis_name="s")
    dispatch = pl.kernel(
        _dispatch_body,
        out_type=jax.ShapeDtypeStruct((PAD, C), jnp.float32),
        mesh=mesh,
        scratch_types=[
            pltpu.VMEM((TPW,), jnp.int32),
            pltpu.VMEM((TPW,), jnp.int32),
            pltpu.VMEM((TPW, C), jnp.float32),
            pltpu.SemaphoreType.DMA,
            pltpu.SemaphoreType.DMA,
        ],
    )
    combine = pl.kernel(
        _combine_body,
        out_type=jax.ShapeDtypeStruct((T, C), jnp.float32),
        mesh=mesh,
        scratch_types=[
            pltpu.VMEM((TPW,), jnp.int32),
            pltpu.VMEM((TPW,), jnp.int32),
            pltpu.VMEM((TPW, C), jnp.float32),
            pltpu.VMEM((TPW, C), jnp.float32),
            pltpu.VMEM((TPW, 16), jnp.float32),
            pltpu.VMEM((TPW, 16), jnp.float32),
            pltpu.SemaphoreType.DMA,
            pltpu.SemaphoreType.DMA,
        ],
    )
    return dispatch, combine


def kernel(x, W_route, b_route, W_noise, b_noise, W1, b1, W2, b2, ln_g, ln_b,
           noise):
    xf = x.reshape(T, C)
    nz = noise.reshape(T, E)

    pos0, pos1, g0r, g1r, bexp, cumblk = pl.pallas_call(
        _router_body,
        out_shape=[
            jax.ShapeDtypeStruct((T, 1), jnp.int32),
            jax.ShapeDtypeStruct((T, 1), jnp.int32),
            jax.ShapeDtypeStruct((T, 16), jnp.float32),
            jax.ShapeDtypeStruct((T, 16), jnp.float32),
            jax.ShapeDtypeStruct((128, 1), jnp.int32),
            jax.ShapeDtypeStruct((1, E), jnp.int32),
        ],
    )(xf, W_route, b_route.reshape(1, E), W_noise, b_noise.reshape(1, E), nz)

    p0w = pos0.reshape(NW, TPW)
    p1w = pos1.reshape(NW, TPW)
    g0w = g0r.reshape(NW, TPW, 16)
    g1w = g1r.reshape(NW, TPW, 16)
    sc = jnp.concatenate([bexp.reshape(128)[:NB],
                          cumblk.reshape(E)[E - 1:]])

    dispatch, combine = _sc_kernels()
    xg = dispatch(xf, p0w, p1w)

    yln = pl.pallas_call(
        _ffn_outer,
        in_specs=[
            pl.BlockSpec(memory_space=pltpu.SMEM),
            pl.BlockSpec(memory_space=pl.ANY),
            pl.BlockSpec(memory_space=pl.ANY),
            pl.BlockSpec(memory_space=pl.ANY),
            pl.BlockSpec(memory_space=pl.ANY),
            pl.BlockSpec(memory_space=pl.ANY),
            pl.BlockSpec(memory_space=pl.ANY),
            pl.BlockSpec(memory_space=pl.ANY),
        ],
        out_specs=pl.BlockSpec(memory_space=pl.ANY),
        out_shape=jax.ShapeDtypeStruct((PAD, C), jnp.float32),
    )(sc, xg, W1, b1.reshape(E, 1, H), W2, b2.reshape(E, 1, C),
      ln_g.reshape(E, 1, C), ln_b.reshape(E, 1, C))

    out = combine(yln, p0w, p1w, g0w, g1w)
    return out.reshape(1, T, C)


# speedup vs baseline: 1.4003x; 1.4003x over previous
import functools

import jax
import jax.numpy as jnp
from jax import lax
from jax.experimental import pallas as pl
from jax.experimental.pallas import tpu as pltpu
from jax.experimental.pallas import tpu_sc as plsc

T, C, E = 2048, 768, 8
H = 4 * C
RB = 256
NB = (2 * T) // RB + E
PAD = NB * RB
NW = 32
TPW = T // NW
GRP = 128
NG = T // GRP
NEG = -3.0e38


def _router_body(x_ref, wr_ref, br_ref, wn_ref, bn_ref, nz_ref,
                 pos0_ref, pos1_ref, g0_ref, g1_ref, bexp_ref, cb_ref):
    xf = x_ref[...]
    nt = (((1,), (1,)), ((), ()))
    logits = lax.dot_general(xf, wr_ref[...], nt,
                             preferred_element_type=jnp.float32) + br_ref[...]
    nlog = lax.dot_general(xf, wn_ref[...], nt,
                           preferred_element_type=jnp.float32) + bn_ref[...]
    noisy = logits + nz_ref[...] * jax.nn.softplus(nlog)

    ia = lax.broadcasted_iota(jnp.int32, (E, E), 0)
    ib = lax.broadcasted_iota(jnp.int32, (E, E), 1)
    u8s = (ia < ib).astype(jnp.float32)
    u8i = (ia <= ib).astype(jnp.float32)
    nn = (((1,), (0,)), ((), ()))

    v0 = jnp.max(noisy, axis=1, keepdims=True)
    eq0 = (noisy == v0).astype(jnp.float32)
    p0 = lax.dot_general(eq0, u8s, nn, preferred_element_type=jnp.float32)
    oh0 = eq0 * (p0 == 0).astype(jnp.float32)
    masked = jnp.where(oh0 > 0, NEG, noisy)
    v1 = jnp.max(masked, axis=1, keepdims=True)
    eq1 = (masked == v1).astype(jnp.float32)
    p1 = lax.dot_general(eq1, u8s, nn, preferred_element_type=jnp.float32)
    oh1 = eq1 * (p1 == 0).astype(jnp.float32)

    e1g = jnp.exp(v1 - v0)
    den = 1.0 + e1g
    g0 = 1.0 / den
    g1 = e1g / den

    msk = oh0 + oh1
    ri = lax.broadcasted_iota(jnp.int32, (GRP, GRP), 0)
    rj = lax.broadcasted_iota(jnp.int32, (GRP, GRP), 1)
    lg = (rj < ri).astype(jnp.float32)
    acc = jnp.zeros((1, E), dtype=jnp.float32)
    parts = []
    for g in range(NG):
        mg = msk[g * GRP:(g + 1) * GRP]
        pg = lax.dot_general(lg, mg, nn, preferred_element_type=jnp.float32)
        parts.append(pg + acc)
        acc = acc + pg[GRP - 1:GRP] + mg[GRP - 1:GRP]
    rank = jnp.concatenate(parts, axis=0)
    counts = acc

    nblk = jnp.floor((counts + (RB - 1)) * (1.0 / RB))
    cumb = lax.dot_general(nblk, u8i, nn,
                           preferred_element_type=jnp.float32)
    offs = (cumb - nblk) * float(RB)

    posf = rank + offs
    pos0_ref[...] = jnp.sum(posf * oh0, axis=1, keepdims=True).astype(jnp.int32)
    pos1_ref[...] = jnp.sum(posf * oh1, axis=1, keepdims=True).astype(jnp.int32)
    g0_ref[...] = jnp.broadcast_to(g0, (T, 16))
    g1_ref[...] = jnp.broadcast_to(g1, (T, 16))

    biota = lax.broadcasted_iota(jnp.int32, (128, E), 0)
    cumb_i = cumb.astype(jnp.int32)
    bexp = jnp.sum((cumb_i <= biota).astype(jnp.int32), axis=1, keepdims=True)
    bexp_ref[...] = jnp.minimum(bexp, E - 1)
    cb_ref[...] = cumb_i


def _ffn_outer(sc_ref, xg_hbm, w1_hbm, b1_hbm, w2_hbm, b2_hbm, lg_hbm,
               lb_hbm, yln_hbm):
    nbtot = sc_ref[NB]

    def inner(xg_ref, w1_ref, b1_ref, w2_ref, b2_ref, lg_ref, lb_ref,
              out_ref):
        nt = (((1,), (1,)), ((), ()))
        xb = xg_ref[...]
        h = lax.dot_general(xb, w1_ref[0], nt,
                            preferred_element_type=jnp.float32)
        h = h + b1_ref[0]
        h = 0.5 * h * (1.0 + lax.erf(h * 0.7071067811865476))
        y = lax.dot_general(h, w2_ref[0], nt,
                            preferred_element_type=jnp.float32)
        z = xb + (y + b2_ref[0])
        mu = jnp.mean(z, axis=1, keepdims=True)
        zc = z - mu
        var = jnp.mean(zc * zc, axis=1, keepdims=True)
        out_ref[...] = zc * lax.rsqrt(var + 1e-5) * lg_ref[0] + lb_ref[0]

    wbuf = pl.Buffered(buffer_count=2, use_lookahead=True)
    wbuf3 = pl.Buffered(buffer_count=3, use_lookahead=True)
    pltpu.emit_pipeline(
        inner,
        grid=(nbtot,),
        in_specs=[
            pl.BlockSpec((RB, C), lambda i: (i, 0)),
            pl.BlockSpec((1, H, C), lambda i: (sc_ref[i], 0, 0),
                         pipeline_mode=wbuf3),
            pl.BlockSpec((1, 1, H), lambda i: (sc_ref[i], 0, 0)),
            pl.BlockSpec((1, C, H), lambda i: (sc_ref[i], 0, 0),
                         pipeline_mode=wbuf),
            pl.BlockSpec((1, 1, C), lambda i: (sc_ref[i], 0, 0)),
            pl.BlockSpec((1, 1, C), lambda i: (sc_ref[i], 0, 0)),
            pl.BlockSpec((1, 1, C), lambda i: (sc_ref[i], 0, 0)),
        ],
        out_specs=[pl.BlockSpec((RB, C), lambda i: (i, 0))],
    )(xg_hbm, w1_hbm, b1_hbm, w2_hbm, b2_hbm, lg_hbm, lb_hbm, yln_hbm)


def _dispatch_body(x_hbm, p0_hbm, p1_hbm, xg_hbm, idx0, idx1, rows, s0, s1):
    wid = lax.axis_index("s") * 2 + lax.axis_index("c")
    base = wid * TPW
    pltpu.sync_copy(p0_hbm.at[wid], idx0)
    pltpu.sync_copy(p1_hbm.at[wid], idx1)
    pltpu.sync_copy(x_hbm.at[pl.ds(base, TPW)], rows)
    d0 = pltpu.async_copy(rows, xg_hbm.at[idx0], s0)
    d1 = pltpu.async_copy(rows, xg_hbm.at[idx1], s1)
    d0.wait()
    d1.wait()


def _combine_body(yln_hbm, p0_hbm, p1_hbm, g0_hbm, g1_hbm, out_hbm,
                  idx0, idx1, r0, r1, gv0, gv1, s0, s1):
    wid = lax.axis_index("s") * 2 + lax.axis_index("c")
    base = wid * TPW
    pltpu.sync_copy(p0_hbm.at[wid], idx0)
    pltpu.sync_copy(p1_hbm.at[wid], idx1)
    pltpu.sync_copy(g0_hbm.at[wid], gv0)
    pltpu.sync_copy(g1_hbm.at[wid], gv1)
    d0 = pltpu.async_copy(yln_hbm.at[idx0], r0, s0)
    d1 = pltpu.async_copy(yln_hbm.at[idx1], r1, s1)
    d0.wait()
    d1.wait()

    def tok(i, carry):
        ga = gv0[i]
        gb = gv1[i]
        for cc in range(C // 16):
            sl = pl.ds(cc * 16, 16)
            r0[i, sl] = ga * r0[i, sl] + gb * r1[i, sl]
        return carry

    lax.fori_loop(0, TPW, tok, 0)
    pltpu.sync_copy(r0, out_hbm.at[pl.ds(base, TPW)])


@functools.lru_cache(maxsize=1)
def _sc_kernels():
    mesh = plsc.VectorSubcoreMesh(core_axis_name="c", subcore_axis_name="s")
    dispatch = pl.kernel(
        _dispatch_body,
        out_type=jax.ShapeDtypeStruct((PAD, C), jnp.float32),
        mesh=mesh,
        scratch_types=[
            pltpu.VMEM((TPW,), jnp.int32),
            pltpu.VMEM((TPW,), jnp.int32),
            pltpu.VMEM((TPW, C), jnp.float32),
            pltpu.SemaphoreType.DMA,
            pltpu.SemaphoreType.DMA,
        ],
    )
    combine = pl.kernel(
        _combine_body,
        out_type=jax.ShapeDtypeStruct((T, C), jnp.float32),
        mesh=mesh,
        scratch_types=[
            pltpu.VMEM((TPW,), jnp.int32),
            pltpu.VMEM((TPW,), jnp.int32),
            pltpu.VMEM((TPW, C), jnp.float32),
            pltpu.VMEM((TPW, C), jnp.float32),
            pltpu.VMEM((TPW, 16), jnp.float32),
            pltpu.VMEM((TPW, 16), jnp.float32),
            pltpu.SemaphoreType.DMA,
            pltpu.SemaphoreType.DMA,
        ],
    )
    return dispatch, combine


def kernel(x, W_route, b_route, W_noise, b_noise, W1, b1, W2, b2, ln_g, ln_b,
           noise):
    xf = x.reshape(T, C)
    nz = noise.reshape(T, E)

    pos0, pos1, g0r, g1r, bexp, cumblk = pl.pallas_call(
        _router_body,
        out_shape=[
            jax.ShapeDtypeStruct((T, 1), jnp.int32),
            jax.ShapeDtypeStruct((T, 1), jnp.int32),
            jax.ShapeDtypeStruct((T, 16), jnp.float32),
            jax.ShapeDtypeStruct((T, 16), jnp.float32),
            jax.ShapeDtypeStruct((128, 1), jnp.int32),
            jax.ShapeDtypeStruct((1, E), jnp.int32),
        ],
    )(xf, W_route, b_route.reshape(1, E), W_noise, b_noise.reshape(1, E), nz)

    p0w = pos0.reshape(NW, TPW)
    p1w = pos1.reshape(NW, TPW)
    g0w = g0r.reshape(NW, TPW, 16)
    g1w = g1r.reshape(NW, TPW, 16)
    sc = jnp.concatenate([bexp.reshape(128)[:NB],
                          cumblk.reshape(E)[E - 1:]])

    dispatch, combine = _sc_kernels()
    xg = dispatch(xf, p0w, p1w)

    yln = pl.pallas_call(
        _ffn_outer,
        in_specs=[
            pl.BlockSpec(memory_space=pltpu.SMEM),
            pl.BlockSpec(memory_space=pl.ANY),
            pl.BlockSpec(memory_space=pl.ANY),
            pl.BlockSpec(memory_space=pl.ANY),
            pl.BlockSpec(memory_space=pl.ANY),
            pl.BlockSpec(memory_space=pl.ANY),
            pl.BlockSpec(memory_space=pl.ANY),
            pl.BlockSpec(memory_space=pl.ANY),
        ],
        out_specs=pl.BlockSpec(memory_space=pl.ANY),
        out_shape=jax.ShapeDtypeStruct((PAD, C), jnp.float32),
    )(sc, xg, W1, b1.reshape(E, 1, H), W2, b2.reshape(E, 1, C),
      ln_g.reshape(E, 1, C), ln_b.reshape(E, 1, C))

    out = combine(yln, p0w, p1w, g0w, g1w)
    return out.reshape(1, T, C)


# speedup vs baseline: 1.4349x; 1.0247x over previous
import functools

import jax
import jax.numpy as jnp
from jax import lax
from jax.experimental import pallas as pl
from jax.experimental.pallas import tpu as pltpu
from jax.experimental.pallas import tpu_sc as plsc

T, C, E = 2048, 768, 8
H = 4 * C
RB = 256
NB = (2 * T) // RB + E
PAD = NB * RB
NW = 32
TPW = T // NW
GRP = 128
NG = T // GRP
NEG = -3.0e38


def _router_body(x_ref, wr_ref, br_ref, wn_ref, bn_ref, nz_ref,
                 pos0_ref, pos1_ref, g0_ref, g1_ref, bexp_ref, cb_ref):
    xf = x_ref[...]
    nt = (((1,), (1,)), ((), ()))
    logits = lax.dot_general(xf, wr_ref[...], nt,
                             preferred_element_type=jnp.float32) + br_ref[...]
    nlog = lax.dot_general(xf, wn_ref[...], nt,
                           preferred_element_type=jnp.float32) + bn_ref[...]
    noisy = logits + nz_ref[...] * jax.nn.softplus(nlog)

    ia = lax.broadcasted_iota(jnp.int32, (E, E), 0)
    ib = lax.broadcasted_iota(jnp.int32, (E, E), 1)
    u8s = (ia < ib).astype(jnp.float32)
    u8i = (ia <= ib).astype(jnp.float32)
    nn = (((1,), (0,)), ((), ()))

    v0 = jnp.max(noisy, axis=1, keepdims=True)
    eq0 = (noisy == v0).astype(jnp.float32)
    p0 = lax.dot_general(eq0, u8s, nn, preferred_element_type=jnp.float32)
    oh0 = eq0 * (p0 == 0).astype(jnp.float32)
    masked = jnp.where(oh0 > 0, NEG, noisy)
    v1 = jnp.max(masked, axis=1, keepdims=True)
    eq1 = (masked == v1).astype(jnp.float32)
    p1 = lax.dot_general(eq1, u8s, nn, preferred_element_type=jnp.float32)
    oh1 = eq1 * (p1 == 0).astype(jnp.float32)

    e1g = jnp.exp(v1 - v0)
    den = 1.0 + e1g
    g0 = 1.0 / den
    g1 = e1g / den

    msk = oh0 + oh1
    ri = lax.broadcasted_iota(jnp.int32, (GRP, GRP), 0)
    rj = lax.broadcasted_iota(jnp.int32, (GRP, GRP), 1)
    lg = (rj < ri).astype(jnp.float32)
    acc = jnp.zeros((1, E), dtype=jnp.float32)
    parts = []
    for g in range(NG):
        mg = msk[g * GRP:(g + 1) * GRP]
        pg = lax.dot_general(lg, mg, nn, preferred_element_type=jnp.float32)
        parts.append(pg + acc)
        acc = acc + pg[GRP - 1:GRP] + mg[GRP - 1:GRP]
    rank = jnp.concatenate(parts, axis=0)
    counts = acc

    nblk = jnp.floor((counts + (RB - 1)) * (1.0 / RB))
    cumb = lax.dot_general(nblk, u8i, nn,
                           preferred_element_type=jnp.float32)
    offs = (cumb - nblk) * float(RB)

    posf = rank + offs
    pos0_ref[...] = jnp.sum(posf * oh0, axis=1, keepdims=True).astype(jnp.int32)
    pos1_ref[...] = jnp.sum(posf * oh1, axis=1, keepdims=True).astype(jnp.int32)
    g0_ref[...] = jnp.broadcast_to(g0, (T, 16))
    g1_ref[...] = jnp.broadcast_to(g1, (T, 16))

    biota = lax.broadcasted_iota(jnp.int32, (128, E), 0)
    cumb_i = cumb.astype(jnp.int32)
    bexp = jnp.sum((cumb_i <= biota).astype(jnp.int32), axis=1, keepdims=True)
    bexp_ref[...] = jnp.minimum(bexp, E - 1)
    cb_ref[...] = cumb_i


def _ffn_outer(sc_ref, xg_hbm, w1_hbm, b1_hbm, w2_hbm, b2_hbm, lg_hbm,
               lb_hbm, yln_hbm):
    nbtot = sc_ref[NB]

    def inner(xg_ref, w1_ref, b1_ref, w2_ref, b2_ref, lg_ref, lb_ref,
              out_ref):
        nt = (((1,), (1,)), ((), ()))
        xb = xg_ref[...]
        h = lax.dot_general(xb, w1_ref[0], nt,
                            preferred_element_type=jnp.float32)
        h = h + b1_ref[0]
        h = 0.5 * h * (1.0 + lax.erf(h * 0.7071067811865476))
        y = lax.dot_general(h, w2_ref[0], nt,
                            preferred_element_type=jnp.float32)
        z = xb + (y + b2_ref[0])
        mu = jnp.mean(z, axis=1, keepdims=True)
        zc = z - mu
        var = jnp.mean(zc * zc, axis=1, keepdims=True)
        out_ref[...] = zc * lax.rsqrt(var + 1e-5) * lg_ref[0] + lb_ref[0]

    wbuf = pl.Buffered(buffer_count=2, use_lookahead=True)
    pltpu.emit_pipeline(
        inner,
        grid=(nbtot,),
        in_specs=[
            pl.BlockSpec((RB, C), lambda i: (i, 0),
                         pipeline_mode=pl.Buffered(buffer_count=3,
                                                   use_lookahead=True)),
            pl.BlockSpec((1, H, C), lambda i: (sc_ref[i], 0, 0),
                         pipeline_mode=wbuf),
            pl.BlockSpec((1, 1, H), lambda i: (sc_ref[i], 0, 0)),
            pl.BlockSpec((1, C, H), lambda i: (sc_ref[i], 0, 0),
                         pipeline_mode=wbuf),
            pl.BlockSpec((1, 1, C), lambda i: (sc_ref[i], 0, 0)),
            pl.BlockSpec((1, 1, C), lambda i: (sc_ref[i], 0, 0)),
            pl.BlockSpec((1, 1, C), lambda i: (sc_ref[i], 0, 0)),
        ],
        out_specs=[pl.BlockSpec((RB, C), lambda i: (i, 0))],
    )(xg_hbm, w1_hbm, b1_hbm, w2_hbm, b2_hbm, lg_hbm, lb_hbm, yln_hbm)


def _dispatch_body(x_hbm, p0_hbm, p1_hbm, xg_hbm, idx0, idx1, rows, s0, s1):
    wid = lax.axis_index("s") * 2 + lax.axis_index("c")
    base = wid * TPW
    pltpu.sync_copy(p0_hbm.at[wid], idx0)
    pltpu.sync_copy(p1_hbm.at[wid], idx1)
    pltpu.sync_copy(x_hbm.at[pl.ds(base, TPW)], rows)
    d0 = pltpu.async_copy(rows, xg_hbm.at[idx0], s0)
    d1 = pltpu.async_copy(rows, xg_hbm.at[idx1], s1)
    d0.wait()
    d1.wait()


def _combine_body(yln_hbm, p0_hbm, p1_hbm, g0_hbm, g1_hbm, out_hbm,
                  idx0, idx1, r0, r1, gv0, gv1, s0, s1):
    wid = lax.axis_index("s") * 2 + lax.axis_index("c")
    base = wid * TPW
    pltpu.sync_copy(p0_hbm.at[wid], idx0)
    pltpu.sync_copy(p1_hbm.at[wid], idx1)
    pltpu.sync_copy(g0_hbm.at[wid], gv0)
    pltpu.sync_copy(g1_hbm.at[wid], gv1)
    d0 = pltpu.async_copy(yln_hbm.at[idx0], r0, s0)
    d1 = pltpu.async_copy(yln_hbm.at[idx1], r1, s1)
    d0.wait()
    d1.wait()

    def tok(i, carry):
        ga = gv0[i]
        gb = gv1[i]
        for cc in range(C // 16):
            sl = pl.ds(cc * 16, 16)
            r0[i, sl] = ga * r0[i, sl] + gb * r1[i, sl]
        return carry

    lax.fori_loop(0, TPW, tok, 0)
    pltpu.sync_copy(r0, out_hbm.at[pl.ds(base, TPW)])


@functools.lru_cache(maxsize=1)
def _sc_kernels():
    mesh = plsc.VectorSubcoreMesh(core_axis_name="c", subcore_axis_name="s")
    dispatch = pl.kernel(
        _dispatch_body,
        out_type=jax.ShapeDtypeStruct((PAD, C), jnp.float32),
        mesh=mesh,
        scratch_types=[
            pltpu.VMEM((TPW,), jnp.int32),
            pltpu.VMEM((TPW,), jnp.int32),
            pltpu.VMEM((TPW, C), jnp.float32),
            pltpu.SemaphoreType.DMA,
            pltpu.SemaphoreType.DMA,
        ],
    )
    combine = pl.kernel(
        _combine_body,
        out_type=jax.ShapeDtypeStruct((T, C), jnp.float32),
        mesh=mesh,
        scratch_types=[
            pltpu.VMEM((TPW,), jnp.int32),
            pltpu.VMEM((TPW,), jnp.int32),
            pltpu.VMEM((TPW, C), jnp.float32),
            pltpu.VMEM((TPW, C), jnp.float32),
            pltpu.VMEM((TPW, 16), jnp.float32),
            pltpu.VMEM((TPW, 16), jnp.float32),
            pltpu.SemaphoreType.DMA,
            pltpu.SemaphoreType.DMA,
        ],
    )
    return dispatch, combine


def kernel(x, W_route, b_route, W_noise, b_noise, W1, b1, W2, b2, ln_g, ln_b,
           noise):
    xf = x.reshape(T, C)
    nz = noise.reshape(T, E)

    pos0, pos1, g0r, g1r, bexp, cumblk = pl.pallas_call(
        _router_body,
        out_shape=[
            jax.ShapeDtypeStruct((T, 1), jnp.int32),
            jax.ShapeDtypeStruct((T, 1), jnp.int32),
            jax.ShapeDtypeStruct((T, 16), jnp.float32),
            jax.ShapeDtypeStruct((T, 16), jnp.float32),
            jax.ShapeDtypeStruct((128, 1), jnp.int32),
            jax.ShapeDtypeStruct((1, E), jnp.int32),
        ],
    )(xf, W_route, b_route.reshape(1, E), W_noise, b_noise.reshape(1, E), nz)

    p0w = pos0.reshape(NW, TPW)
    p1w = pos1.reshape(NW, TPW)
    g0w = g0r.reshape(NW, TPW, 16)
    g1w = g1r.reshape(NW, TPW, 16)
    sc = jnp.concatenate([bexp.reshape(128)[:NB],
                          cumblk.reshape(E)[E - 1:]])

    dispatch, combine = _sc_kernels()
    xg = dispatch(xf, p0w, p1w)

    yln = pl.pallas_call(
        _ffn_outer,
        in_specs=[
            pl.BlockSpec(memory_space=pltpu.SMEM),
            pl.BlockSpec(memory_space=pl.ANY),
            pl.BlockSpec(memory_space=pl.ANY),
            pl.BlockSpec(memory_space=pl.ANY),
            pl.BlockSpec(memory_space=pl.ANY),
            pl.BlockSpec(memory_space=pl.ANY),
            pl.BlockSpec(memory_space=pl.ANY),
            pl.BlockSpec(memory_space=pl.ANY),
        ],
        out_specs=pl.BlockSpec(memory_space=pl.ANY),
        out_shape=jax.ShapeDtypeStruct((PAD, C), jnp.float32),
    )(sc, xg, W1, b1.reshape(E, 1, H), W2, b2.reshape(E, 1, C),
      ln_g.reshape(E, 1, C), ln_b.reshape(E, 1, C))

    out = combine(yln, p0w, p1w, g0w, g1w)
    return out.reshape(1, T, C)
